# baseline (speedup 1.0000x reference)
import functools

import jax
import jax.numpy as jnp
from jax import lax
from jax.experimental import pallas as pl
from jax.experimental.pallas import tpu as pltpu
from jax.experimental.pallas import tpu_sc as plsc

N_NODES = 2048
N_EDGES = 131072
HIDDEN = 128
HEADS = 8
HEAD_DIM = HIDDEN // HEADS

NUM_SC = 2
NUM_TILES = 16
EDGES_PER_TILE = N_EDGES // NUM_TILES
ROWS_PER_PASS = 512
PASSES = (N_NODES // NUM_SC) // ROWS_PER_PASS
BUF = ROWS_PER_PASS * N_NODES
WORDS_PER_TILE = BUF // NUM_TILES
CHUNK = 128
N_CHUNKS = EDGES_PER_TILE // CHUNK
ZBUF = 16384


@functools.lru_cache(maxsize=1)
def _make_build_adjacency():
    return functools.partial(
        pl.kernel,
        out_type=jax.ShapeDtypeStruct((N_NODES * N_NODES,), jnp.float32),
        mesh=plsc.VectorSubcoreMesh(core_axis_name="c", subcore_axis_name="s"),
        scratch_types=[
            pltpu.VMEM_SHARED((BUF,), jnp.float32),
            pltpu.VMEM((EDGES_PER_TILE,), jnp.int32),
            pltpu.VMEM((EDGES_PER_TILE,), jnp.int32),
            pltpu.VMEM((EDGES_PER_TILE,), jnp.int32),
            pltpu.VMEM((EDGES_PER_TILE,), jnp.float32),
            pltpu.VMEM((ZBUF,), jnp.float32),
        ],
    )(_build_adjacency_body)


def _build_adjacency_body(edge_hbm, a_hbm, shared, dstv, srcv, idxv, valv, zerosv):
    c = lax.axis_index("c")
    s = lax.axis_index("s")

    base = s * EDGES_PER_TILE
    pltpu.sync_copy(edge_hbm.at[pl.ds(base, EDGES_PER_TILE)], dstv)
    pltpu.sync_copy(edge_hbm.at[pl.ds(N_EDGES + base, EDGES_PER_TILE)], srcv)

    def _zfill(i, _):
        for u in range(4):
            zerosv[pl.ds(i * 64 + u * 16, 16)] = jnp.zeros((16,), jnp.float32)
        return 0
    lax.fori_loop(0, ZBUF // 64, _zfill, 0)

    def _ifill(j, _):
        for u in range(4):
            off = j * 64 + u * 16
            d = dstv[pl.ds(off, 16)]
            sidx = srcv[pl.ds(off, 16)]
            idxv[pl.ds(off, 16)] = (d & (ROWS_PER_PASS - 1)) * N_NODES + sidx
        return 0
    lax.fori_loop(0, EDGES_PER_TILE // 64, _ifill, 0)

    for p in range(PASSES):
        row0 = c * (N_NODES // NUM_SC) + p * ROWS_PER_PASS

        for z in range(WORDS_PER_TILE // ZBUF):
            pltpu.sync_copy(
                zerosv,
                shared.at[pl.ds(s * WORDS_PER_TILE + z * ZBUF, ZBUF)],
            )

        def _vfill(j, _):
            for u in range(4):
                off = j * 64 + u * 16
                d = dstv[pl.ds(off, 16)]
                ok = (d >= row0) & (d < row0 + ROWS_PER_PASS)
                valv[pl.ds(off, 16)] = jnp.where(ok, 1.0, 0.0).astype(jnp.float32)
            return 0
        lax.fori_loop(0, EDGES_PER_TILE // 64, _vfill, 0)
        plsc.subcore_barrier()

        pltpu.sync_copy(valv, shared.at[idxv], add=True)
        plsc.subcore_barrier()

        pltpu.sync_copy(
            shared.at[pl.ds(s * WORDS_PER_TILE, WORDS_PER_TILE)],
            a_hbm.at[pl.ds(row0 * N_NODES + s * WORDS_PER_TILE, WORDS_PER_TILE)],
        )


BM = 256
GRID = N_NODES // BM


LAYERS = 3
STEPS_PER_LAYER = 2 * GRID
N_STEPS = LAYERS * STEPS_PER_LAYER + 1


def _fused_body(a_ref, h0_ref, w_ref, b_ref, inw_ref, inb_ref, ow_ref, ob_ref,
                pw_ref, pb_ref, out_ref, qkv_scr, h_scr):
    g = pl.program_id(0)
    phase = (g // GRID) % 2
    blk = g % GRID
    row = blk * BM
    scale = 1.0 / (HEAD_DIM ** 0.5)

    @pl.when(g == 0)
    def _init():
        h_scr[...] = h0_ref[...]

    @pl.when((g < LAYERS * STEPS_PER_LAYER) & (phase == 0))
    def _qkv_step():
        a = a_ref[...]
        u = jnp.dot(a, h_scr[...], preferred_element_type=jnp.float32)
        ones_col = jnp.ones((N_NODES, 8), jnp.float32)
        deg = jnp.dot(a, ones_col, preferred_element_type=jnp.float32)[:, :1]
        hn = lax.dot_general(u, w_ref[0], (((1,), (1,)), ((), ())),
                             preferred_element_type=jnp.float32) + deg * b_ref[0]
        qkv_scr[pl.ds(row, BM), :] = lax.dot_general(
            hn, inw_ref[...], (((1,), (1,)), ((), ())),
            preferred_element_type=jnp.float32) + inb_ref[...]

    @pl.when((g < LAYERS * STEPS_PER_LAYER) & (phase == 1))
    def _attn_step():
        kv_all = qkv_scr[:, HIDDEN:]
        qkv_q = qkv_scr[pl.ds(row, BM), :]
        outs = []
        for hh in range(HEADS):
            lo = hh * HEAD_DIM
            q = qkv_q[:, lo:lo + HEAD_DIM]
            k = kv_all[:, lo:lo + HEAD_DIM]
            v = kv_all[:, HIDDEN + lo:HIDDEN + lo + HEAD_DIM]
            sc = lax.dot_general(q, k, (((1,), (1,)), ((), ())),
                                 preferred_element_type=jnp.float32) * scale
            m = jnp.max(sc, axis=1, keepdims=True)
            e = jnp.exp(sc - m)
            denom = jnp.sum(e, axis=1, keepdims=True)
            p = e / denom
            outs.append(jnp.dot(p, v, preferred_element_type=jnp.float32))
        att = jnp.concatenate(outs, axis=1)
        att = lax.dot_general(att, ow_ref[...], (((1,), (1,)), ((), ())),
                              preferred_element_type=jnp.float32) + ob_ref[...]
        h_scr[pl.ds(row, BM), :] = jnp.maximum(att + h_scr[pl.ds(row, BM), :], 0.0)

    @pl.when(g == N_STEPS - 1)
    def _proj_step():
        z = jnp.dot(h_scr[...], pw_ref[...],
                    preferred_element_type=jnp.float32)[:, :1] + pb_ref[0, 0]
        out_ref[...] = jax.nn.sigmoid(z)


def _fused_layers(A, h0, Wstack, bstack, in_w, in_b, out_w, out_b, pw, pb):
    cmax = LAYERS * STEPS_PER_LAYER

    def a_map(g):
        qkv_phase = ((g // GRID) % 2 == 0) & (g < cmax)
        return (jnp.where(qkv_phase, g % GRID, GRID - 1), 0)

    def w_map(g):
        return (jnp.minimum(g // STEPS_PER_LAYER, LAYERS - 1), 0, 0)

    const2 = lambda g: (0, 0)
    const3 = lambda g: (0, 0, 0)
    return pl.pallas_call(
        _fused_body,
        grid=(N_STEPS,),
        in_specs=[
            pl.BlockSpec((BM, N_NODES), a_map),
            pl.BlockSpec((N_NODES, HIDDEN), const2),
            pl.BlockSpec((1, HIDDEN, HIDDEN), w_map),
            pl.BlockSpec((1, 1, HIDDEN), w_map),
            pl.BlockSpec((3 * HIDDEN, HIDDEN), const2),
            pl.BlockSpec((1, 3 * HIDDEN), const2),
            pl.BlockSpec((HIDDEN, HIDDEN), const2),
            pl.BlockSpec((1, HIDDEN), const2),
            pl.BlockSpec((HIDDEN, 8), const2),
            pl.BlockSpec((1, 1), const2),
        ],
        out_specs=pl.BlockSpec((N_NODES, 1), const2),
        out_shape=jax.ShapeDtypeStruct((N_NODES, 1), jnp.float32),
        scratch_shapes=[
            pltpu.VMEM((N_NODES, 3 * HIDDEN), jnp.float32),
            pltpu.VMEM((N_NODES, HIDDEN), jnp.float32),
        ],
    )(A, h0, Wstack, bstack, in_w, in_b, out_w, out_b, pw, pb)


def kernel(node_embeddings, W0, b0, W1, b1, W2, b2, in_proj_w, in_proj_b,
           out_proj_w, out_proj_b, proj_w, proj_b, edge_index):
    a_flat = _make_build_adjacency()(edge_index.reshape(-1))
    A = a_flat.reshape(N_NODES, N_NODES)
    Wstack = jnp.stack((W0, W1, W2))
    bstack = jnp.stack((b0, b1, b2)).reshape(LAYERS, 1, HIDDEN)
    pw_pad = jnp.concatenate(
        [proj_w.reshape(HIDDEN, 1), jnp.zeros((HIDDEN, 7), jnp.float32)], axis=1)
    return _fused_layers(A, node_embeddings, Wstack, bstack, in_proj_w,
                         in_proj_b.reshape(1, 3 * HIDDEN), out_proj_w,
                         out_proj_b.reshape(1, HIDDEN), pw_pad,
                         proj_b.reshape(1, 1))


# speedup vs baseline: 1.1870x; 1.1870x over previous
import functools

import jax
import jax.numpy as jnp
from jax import lax
from jax.experimental import pallas as pl
from jax.experimental.pallas import tpu as pltpu
from jax.experimental.pallas import tpu_sc as plsc

N_NODES = 2048
N_EDGES = 131072
HIDDEN = 128
HEADS = 8
HEAD_DIM = HIDDEN // HEADS

NUM_SC = 2
NUM_TILES = 16
EDGES_PER_TILE = N_EDGES // NUM_TILES
ROWS_PER_PASS = 512
PASSES = (N_NODES // NUM_SC) // ROWS_PER_PASS
BUF = ROWS_PER_PASS * N_NODES
WORDS_PER_TILE = BUF // NUM_TILES
CHUNK = 128
N_CHUNKS = EDGES_PER_TILE // CHUNK
ZBUF = 16384


@functools.lru_cache(maxsize=1)
def _make_build_adjacency():
    return functools.partial(
        pl.kernel,
        out_type=jax.ShapeDtypeStruct((N_NODES * N_NODES,), jnp.float32),
        mesh=plsc.VectorSubcoreMesh(core_axis_name="c", subcore_axis_name="s"),
        scratch_types=[
            pltpu.VMEM_SHARED((BUF,), jnp.float32),
            pltpu.VMEM((EDGES_PER_TILE,), jnp.int32),
            pltpu.VMEM((EDGES_PER_TILE,), jnp.int32),
            pltpu.VMEM((EDGES_PER_TILE,), jnp.int32),
            pltpu.VMEM((EDGES_PER_TILE,), jnp.float32),
            pltpu.VMEM((ZBUF,), jnp.float32),
        ],
    )(_build_adjacency_body)


def _build_adjacency_body(edge_hbm, a_hbm, shared, dstv, srcv, idxv, valv, zerosv):
    c = lax.axis_index("c")
    s = lax.axis_index("s")

    base = s * EDGES_PER_TILE
    pltpu.sync_copy(edge_hbm.at[pl.ds(base, EDGES_PER_TILE)], dstv)
    pltpu.sync_copy(edge_hbm.at[pl.ds(N_EDGES + base, EDGES_PER_TILE)], srcv)

    def _zfill(i, _):
        for u in range(4):
            zerosv[pl.ds(i * 64 + u * 16, 16)] = jnp.zeros((16,), jnp.float32)
        return 0
    lax.fori_loop(0, ZBUF // 64, _zfill, 0)

    def _ifill(j, _):
        for u in range(4):
            off = j * 64 + u * 16
            d = dstv[pl.ds(off, 16)]
            sidx = srcv[pl.ds(off, 16)]
            idxv[pl.ds(off, 16)] = (d & (ROWS_PER_PASS - 1)) * N_NODES + sidx
        return 0
    lax.fori_loop(0, EDGES_PER_TILE // 64, _ifill, 0)

    for p in range(PASSES):
        row0 = c * (N_NODES // NUM_SC) + p * ROWS_PER_PASS

        for z in range(WORDS_PER_TILE // ZBUF):
            pltpu.sync_copy(
                zerosv,
                shared.at[pl.ds(s * WORDS_PER_TILE + z * ZBUF, ZBUF)],
            )

        def _vfill(j, _):
            for u in range(4):
                off = j * 64 + u * 16
                d = dstv[pl.ds(off, 16)]
                ok = (d >= row0) & (d < row0 + ROWS_PER_PASS)
                valv[pl.ds(off, 16)] = jnp.where(ok, 1.0, 0.0).astype(jnp.float32)
            return 0
        lax.fori_loop(0, EDGES_PER_TILE // 64, _vfill, 0)
        plsc.subcore_barrier()

        pltpu.sync_copy(valv, shared.at[idxv], add=True)
        plsc.subcore_barrier()

        pltpu.sync_copy(
            shared.at[pl.ds(s * WORDS_PER_TILE, WORDS_PER_TILE)],
            a_hbm.at[pl.ds(row0 * N_NODES + s * WORDS_PER_TILE, WORDS_PER_TILE)],
        )


BM = 256
GRID = N_NODES // BM


LAYERS = 3
STEPS_PER_LAYER = 2 * GRID
N_STEPS = LAYERS * STEPS_PER_LAYER + 1


def _fused_body(a_ref, h0_ref, w_ref, b_ref, inw_ref, inb_ref, ow_ref, ob_ref,
                pw_ref, pb_ref, out_ref, qkv_scr, h_scr):
    g = pl.program_id(0)
    phase = (g // GRID) % 2
    blk = g % GRID
    row = blk * BM
    scale = 1.0 / (HEAD_DIM ** 0.5)

    @pl.when(g == 0)
    def _init():
        h_scr[...] = h0_ref[...]

    @pl.when((g < LAYERS * STEPS_PER_LAYER) & (phase == 0))
    def _qkv_step():
        a = a_ref[...]
        u = jnp.dot(a, h_scr[...], preferred_element_type=jnp.float32)
        ones_col = jnp.ones((N_NODES, 8), jnp.float32)
        deg = jnp.dot(a, ones_col, preferred_element_type=jnp.float32)[:, :1]
        hn = lax.dot_general(u, w_ref[0], (((1,), (1,)), ((), ())),
                             preferred_element_type=jnp.float32) + deg * b_ref[0]
        qkv_scr[pl.ds(row, BM), :] = lax.dot_general(
            hn, inw_ref[...], (((1,), (1,)), ((), ())),
            preferred_element_type=jnp.float32) + inb_ref[...]

    @pl.when((g < LAYERS * STEPS_PER_LAYER) & (phase == 1))
    def _attn_step():
        kv_all = qkv_scr[:, HIDDEN:]
        qkv_q = qkv_scr[pl.ds(row, BM), :]
        ones_kcol = jnp.ones((N_NODES, 1), jnp.float32)
        outs = []
        for hh in range(HEADS):
            lo = hh * HEAD_DIM
            q = qkv_q[:, lo:lo + HEAD_DIM]
            k = kv_all[:, lo:lo + HEAD_DIM]
            v = kv_all[:, HIDDEN + lo:HIDDEN + lo + HEAD_DIM]
            sc = lax.dot_general(q, k, (((1,), (1,)), ((), ())),
                                 preferred_element_type=jnp.float32) * scale
            m = jnp.max(sc, axis=1, keepdims=True)
            e = jnp.exp(sc - m)
            nv = jnp.dot(e, jnp.concatenate([v, ones_kcol], axis=1),
                         preferred_element_type=jnp.float32)
            outs.append(nv[:, :HEAD_DIM] / nv[:, HEAD_DIM:HEAD_DIM + 1])
        att = jnp.concatenate(outs, axis=1)
        att = lax.dot_general(att, ow_ref[...], (((1,), (1,)), ((), ())),
                              preferred_element_type=jnp.float32) + ob_ref[...]
        h_scr[pl.ds(row, BM), :] = jnp.maximum(att + h_scr[pl.ds(row, BM), :], 0.0)

    @pl.when(g == N_STEPS - 1)
    def _proj_step():
        z = jnp.dot(h_scr[...], pw_ref[...],
                    preferred_element_type=jnp.float32)[:, :1] + pb_ref[0, 0]
        out_ref[...] = jax.nn.sigmoid(z)


def _fused_layers(A, h0, Wstack, bstack, in_w, in_b, out_w, out_b, pw, pb):
    cmax = LAYERS * STEPS_PER_LAYER

    def a_map(g):
        qkv_phase = ((g // GRID) % 2 == 0) & (g < cmax)
        return (jnp.where(qkv_phase, g % GRID, GRID - 1), 0)

    def w_map(g):
        return (jnp.minimum(g // STEPS_PER_LAYER, LAYERS - 1), 0, 0)

    const2 = lambda g: (0, 0)
    const3 = lambda g: (0, 0, 0)
    return pl.pallas_call(
        _fused_body,
        grid=(N_STEPS,),
        in_specs=[
            pl.BlockSpec((BM, N_NODES), a_map),
            pl.BlockSpec((N_NODES, HIDDEN), const2),
            pl.BlockSpec((1, HIDDEN, HIDDEN), w_map),
            pl.BlockSpec((1, 1, HIDDEN), w_map),
            pl.BlockSpec((3 * HIDDEN, HIDDEN), const2),
            pl.BlockSpec((1, 3 * HIDDEN), const2),
            pl.BlockSpec((HIDDEN, HIDDEN), const2),
            pl.BlockSpec((1, HIDDEN), const2),
            pl.BlockSpec((HIDDEN, 8), const2),
            pl.BlockSpec((1, 1), const2),
        ],
        out_specs=pl.BlockSpec((N_NODES, 1), const2),
        out_shape=jax.ShapeDtypeStruct((N_NODES, 1), jnp.float32),
        scratch_shapes=[
            pltpu.VMEM((N_NODES, 3 * HIDDEN), jnp.float32),
            pltpu.VMEM((N_NODES, HIDDEN), jnp.float32),
        ],
    )(A, h0, Wstack, bstack, in_w, in_b, out_w, out_b, pw, pb)


def kernel(node_embeddings, W0, b0, W1, b1, W2, b2, in_proj_w, in_proj_b,
           out_proj_w, out_proj_b, proj_w, proj_b, edge_index):
    a_flat = _make_build_adjacency()(edge_index.reshape(-1))
    A = a_flat.reshape(N_NODES, N_NODES)
    Wstack = jnp.stack((W0, W1, W2))
    bstack = jnp.stack((b0, b1, b2)).reshape(LAYERS, 1, HIDDEN)
    pw_pad = jnp.concatenate(
        [proj_w.reshape(HIDDEN, 1), jnp.zeros((HIDDEN, 7), jnp.float32)], axis=1)
    return _fused_layers(A, node_embeddings, Wstack, bstack, in_proj_w,
                         in_proj_b.reshape(1, 3 * HIDDEN), out_proj_w,
                         out_proj_b.reshape(1, HIDDEN), pw_pad,
                         proj_b.reshape(1, 1))


# speedup vs baseline: 1.3074x; 1.1014x over previous
import functools

import jax
import jax.numpy as jnp
from jax import lax
from jax.experimental import pallas as pl
from jax.experimental.pallas import tpu as pltpu
from jax.experimental.pallas import tpu_sc as plsc

N_NODES = 2048
N_EDGES = 131072
HIDDEN = 128
HEADS = 8
HEAD_DIM = HIDDEN // HEADS

NUM_SC = 2
NUM_TILES = 16
EDGES_PER_TILE = N_EDGES // NUM_TILES
ROWS_PER_PASS = 512
PASSES = (N_NODES // NUM_SC) // ROWS_PER_PASS
BUF = ROWS_PER_PASS * N_NODES
WORDS_PER_TILE = BUF // NUM_TILES
CHUNK = 128
N_CHUNKS = EDGES_PER_TILE // CHUNK
ZBUF = 16384


@functools.lru_cache(maxsize=1)
def _make_build_adjacency():
    return functools.partial(
        pl.kernel,
        out_type=jax.ShapeDtypeStruct((N_NODES * N_NODES,), jnp.float32),
        mesh=plsc.VectorSubcoreMesh(core_axis_name="c", subcore_axis_name="s"),
        scratch_types=[
            pltpu.VMEM_SHARED((BUF,), jnp.float32),
            pltpu.VMEM((EDGES_PER_TILE,), jnp.int32),
            pltpu.VMEM((EDGES_PER_TILE,), jnp.int32),
            pltpu.VMEM((EDGES_PER_TILE,), jnp.int32),
            pltpu.VMEM((EDGES_PER_TILE,), jnp.float32),
            pltpu.VMEM((ZBUF,), jnp.float32),
        ],
    )(_build_adjacency_body)


def _build_adjacency_body(edge_hbm, a_hbm, shared, dstv, srcv, idxv, valv, zerosv):
    c = lax.axis_index("c")
    s = lax.axis_index("s")

    base = s * EDGES_PER_TILE
    pltpu.sync_copy(edge_hbm.at[pl.ds(base, EDGES_PER_TILE)], dstv)
    pltpu.sync_copy(edge_hbm.at[pl.ds(N_EDGES + base, EDGES_PER_TILE)], srcv)

    def _zfill(i, _):
        for u in range(4):
            zerosv[pl.ds(i * 64 + u * 16, 16)] = jnp.zeros((16,), jnp.float32)
        return 0
    lax.fori_loop(0, ZBUF // 64, _zfill, 0)

    def _ifill(j, _):
        for u in range(4):
            off = j * 64 + u * 16
            d = dstv[pl.ds(off, 16)]
            sidx = srcv[pl.ds(off, 16)]
            idxv[pl.ds(off, 16)] = (d & (ROWS_PER_PASS - 1)) * N_NODES + sidx
        return 0
    lax.fori_loop(0, EDGES_PER_TILE // 64, _ifill, 0)

    for p in range(PASSES):
        row0 = c * (N_NODES // NUM_SC) + p * ROWS_PER_PASS

        for z in range(WORDS_PER_TILE // ZBUF):
            pltpu.sync_copy(
                zerosv,
                shared.at[pl.ds(s * WORDS_PER_TILE + z * ZBUF, ZBUF)],
            )

        def _vfill(j, _):
            for u in range(4):
                off = j * 64 + u * 16
                d = dstv[pl.ds(off, 16)]
                ok = (d >= row0) & (d < row0 + ROWS_PER_PASS)
                valv[pl.ds(off, 16)] = jnp.where(ok, 1.0, 0.0).astype(jnp.float32)
            return 0
        lax.fori_loop(0, EDGES_PER_TILE // 64, _vfill, 0)
        plsc.subcore_barrier()

        pltpu.sync_copy(valv, shared.at[idxv], add=True)
        plsc.subcore_barrier()

        pltpu.sync_copy(
            shared.at[pl.ds(s * WORDS_PER_TILE, WORDS_PER_TILE)],
            a_hbm.at[pl.ds(row0 * N_NODES + s * WORDS_PER_TILE, WORDS_PER_TILE)],
        )


BM = 512
GRID = N_NODES // BM


LAYERS = 3
STEPS_PER_LAYER = 2 * GRID
N_STEPS = LAYERS * STEPS_PER_LAYER + 1


def _fused_body(a_ref, h0_ref, w_ref, b_ref, inw_ref, inb_ref, ow_ref, ob_ref,
                pw_ref, pb_ref, out_ref, qkv_scr, h_scr):
    g = pl.program_id(0)
    phase = (g // GRID) % 2
    blk = g % GRID
    row = blk * BM
    scale = 1.0 / (HEAD_DIM ** 0.5)

    @pl.when(g == 0)
    def _init():
        h_scr[...] = h0_ref[...]

    @pl.when((g < LAYERS * STEPS_PER_LAYER) & (phase == 0))
    def _qkv_step():
        a = a_ref[...]
        u = jnp.dot(a, h_scr[...], preferred_element_type=jnp.float32)
        ones_col = jnp.ones((N_NODES, 8), jnp.float32)
        deg = jnp.dot(a, ones_col, preferred_element_type=jnp.float32)[:, :1]
        hn = lax.dot_general(u, w_ref[0], (((1,), (1,)), ((), ())),
                             preferred_element_type=jnp.float32) + deg * b_ref[0]
        qkv_scr[pl.ds(row, BM), :] = lax.dot_general(
            hn, inw_ref[...], (((1,), (1,)), ((), ())),
            preferred_element_type=jnp.float32) + inb_ref[...]

    @pl.when((g < LAYERS * STEPS_PER_LAYER) & (phase == 1))
    def _attn_step():
        kv_all = qkv_scr[:, HIDDEN:]
        qkv_q = qkv_scr[pl.ds(row, BM), :]
        ones_kcol = jnp.ones((N_NODES, 1), jnp.float32)
        outs = []
        for hh in range(HEADS):
            lo = hh * HEAD_DIM
            q = qkv_q[:, lo:lo + HEAD_DIM]
            k = kv_all[:, lo:lo + HEAD_DIM]
            v = kv_all[:, HIDDEN + lo:HIDDEN + lo + HEAD_DIM]
            sc = lax.dot_general(q, k, (((1,), (1,)), ((), ())),
                                 preferred_element_type=jnp.float32) * scale
            m = jnp.max(sc, axis=1, keepdims=True)
            e = jnp.exp(sc - m)
            nv = jnp.dot(e, jnp.concatenate([v, ones_kcol], axis=1),
                         preferred_element_type=jnp.float32)
            outs.append(nv[:, :HEAD_DIM] / nv[:, HEAD_DIM:HEAD_DIM + 1])
        att = jnp.concatenate(outs, axis=1)
        att = lax.dot_general(att, ow_ref[...], (((1,), (1,)), ((), ())),
                              preferred_element_type=jnp.float32) + ob_ref[...]
        h_scr[pl.ds(row, BM), :] = jnp.maximum(att + h_scr[pl.ds(row, BM), :], 0.0)

    @pl.when(g == N_STEPS - 1)
    def _proj_step():
        z = jnp.dot(h_scr[...], pw_ref[...],
                    preferred_element_type=jnp.float32)[:, :1] + pb_ref[0, 0]
        out_ref[...] = jax.nn.sigmoid(z)


def _fused_layers(A, h0, Wstack, bstack, in_w, in_b, out_w, out_b, pw, pb):
    cmax = LAYERS * STEPS_PER_LAYER

    def a_map(g):
        qkv_phase = ((g // GRID) % 2 == 0) & (g < cmax)
        return (jnp.where(qkv_phase, g % GRID, GRID - 1), 0)

    def w_map(g):
        return (jnp.minimum(g // STEPS_PER_LAYER, LAYERS - 1), 0, 0)

    const2 = lambda g: (0, 0)
    const3 = lambda g: (0, 0, 0)
    return pl.pallas_call(
        _fused_body,
        grid=(N_STEPS,),
        in_specs=[
            pl.BlockSpec((BM, N_NODES), a_map),
            pl.BlockSpec((N_NODES, HIDDEN), const2),
            pl.BlockSpec((1, HIDDEN, HIDDEN), w_map),
            pl.BlockSpec((1, 1, HIDDEN), w_map),
            pl.BlockSpec((3 * HIDDEN, HIDDEN), const2),
            pl.BlockSpec((1, 3 * HIDDEN), const2),
            pl.BlockSpec((HIDDEN, HIDDEN), const2),
            pl.BlockSpec((1, HIDDEN), const2),
            pl.BlockSpec((HIDDEN, 8), const2),
            pl.BlockSpec((1, 1), const2),
        ],
        out_specs=pl.BlockSpec((N_NODES, 1), const2),
        out_shape=jax.ShapeDtypeStruct((N_NODES, 1), jnp.float32),
        scratch_shapes=[
            pltpu.VMEM((N_NODES, 3 * HIDDEN), jnp.float32),
            pltpu.VMEM((N_NODES, HIDDEN), jnp.float32),
        ],
    )(A, h0, Wstack, bstack, in_w, in_b, out_w, out_b, pw, pb)


def kernel(node_embeddings, W0, b0, W1, b1, W2, b2, in_proj_w, in_proj_b,
           out_proj_w, out_proj_b, proj_w, proj_b, edge_index):
    a_flat = _make_build_adjacency()(edge_index.reshape(-1))
    A = a_flat.reshape(N_NODES, N_NODES)
    Wstack = jnp.stack((W0, W1, W2))
    bstack = jnp.stack((b0, b1, b2)).reshape(LAYERS, 1, HIDDEN)
    pw_pad = jnp.concatenate(
        [proj_w.reshape(HIDDEN, 1), jnp.zeros((HIDDEN, 7), jnp.float32)], axis=1)
    return _fused_layers(A, node_embeddings, Wstack, bstack, in_proj_w,
                         in_proj_b.reshape(1, 3 * HIDDEN), out_proj_w,
                         out_proj_b.reshape(1, HIDDEN), pw_pad,
                         proj_b.reshape(1, 1))


# speedup vs baseline: 1.3143x; 1.0053x over previous
import functools

import jax
import jax.numpy as jnp
from jax import lax
from jax.experimental import pallas as pl
from jax.experimental.pallas import tpu as pltpu
from jax.experimental.pallas import tpu_sc as plsc

N_NODES = 2048
N_EDGES = 131072
HIDDEN = 128
HEADS = 8
HEAD_DIM = HIDDEN // HEADS

NUM_SC = 2
NUM_TILES = 16
EDGES_PER_TILE = N_EDGES // NUM_TILES
ROWS_PER_PASS = 512
PASSES = (N_NODES // NUM_SC) // ROWS_PER_PASS
BUF = ROWS_PER_PASS * N_NODES
WORDS_PER_TILE = BUF // NUM_TILES
CHUNK = 128
N_CHUNKS = EDGES_PER_TILE // CHUNK
ZBUF = 16384


@functools.lru_cache(maxsize=1)
def _make_build_adjacency():
    return functools.partial(
        pl.kernel,
        out_type=jax.ShapeDtypeStruct((N_NODES * N_NODES,), jnp.float32),
        mesh=plsc.VectorSubcoreMesh(core_axis_name="c", subcore_axis_name="s"),
        scratch_types=[
            pltpu.VMEM_SHARED((BUF,), jnp.float32),
            pltpu.VMEM((EDGES_PER_TILE,), jnp.int32),
            pltpu.VMEM((EDGES_PER_TILE,), jnp.int32),
            pltpu.VMEM((EDGES_PER_TILE,), jnp.int32),
            pltpu.VMEM((EDGES_PER_TILE,), jnp.float32),
            pltpu.VMEM((ZBUF,), jnp.float32),
        ],
    )(_build_adjacency_body)


def _build_adjacency_body(edge_hbm, a_hbm, shared, dstv, srcv, idxv, valv, zerosv):
    c = lax.axis_index("c")
    s = lax.axis_index("s")

    base = s * EDGES_PER_TILE
    pltpu.sync_copy(edge_hbm.at[pl.ds(base, EDGES_PER_TILE)], dstv)
    pltpu.sync_copy(edge_hbm.at[pl.ds(N_EDGES + base, EDGES_PER_TILE)], srcv)

    def _zfill(i, _):
        for u in range(4):
            zerosv[pl.ds(i * 64 + u * 16, 16)] = jnp.zeros((16,), jnp.float32)
        return 0
    lax.fori_loop(0, ZBUF // 64, _zfill, 0)

    def _ifill(j, _):
        for u in range(4):
            off = j * 64 + u * 16
            d = dstv[pl.ds(off, 16)]
            sidx = srcv[pl.ds(off, 16)]
            idxv[pl.ds(off, 16)] = (d & (ROWS_PER_PASS - 1)) * N_NODES + sidx
        return 0
    lax.fori_loop(0, EDGES_PER_TILE // 64, _ifill, 0)

    for p in range(PASSES):
        row0 = c * (N_NODES // NUM_SC) + p * ROWS_PER_PASS

        for z in range(WORDS_PER_TILE // ZBUF):
            pltpu.sync_copy(
                zerosv,
                shared.at[pl.ds(s * WORDS_PER_TILE + z * ZBUF, ZBUF)],
            )

        def _vfill(j, _):
            for u in range(4):
                off = j * 64 + u * 16
                d = dstv[pl.ds(off, 16)]
                ok = (d >= row0) & (d < row0 + ROWS_PER_PASS)
                valv[pl.ds(off, 16)] = jnp.where(ok, 1.0, 0.0).astype(jnp.float32)
            return 0
        lax.fori_loop(0, EDGES_PER_TILE // 64, _vfill, 0)
        plsc.subcore_barrier()

        pltpu.sync_copy(valv, shared.at[idxv], add=True)
        plsc.subcore_barrier()

        pltpu.sync_copy(
            shared.at[pl.ds(s * WORDS_PER_TILE, WORDS_PER_TILE)],
            a_hbm.at[pl.ds(row0 * N_NODES + s * WORDS_PER_TILE, WORDS_PER_TILE)],
        )


BM = 1024
GRID = N_NODES // BM


LAYERS = 3
STEPS_PER_LAYER = 2 * GRID
N_STEPS = LAYERS * STEPS_PER_LAYER + 1


def _fused_body(a_ref, h0_ref, w_ref, b_ref, inw_ref, inb_ref, ow_ref, ob_ref,
                pw_ref, pb_ref, out_ref, qkv_scr, h_scr):
    g = pl.program_id(0)
    phase = (g // GRID) % 2
    blk = g % GRID
    row = blk * BM
    scale = 1.0 / (HEAD_DIM ** 0.5)

    @pl.when(g == 0)
    def _init():
        h_scr[...] = h0_ref[...]

    @pl.when((g < LAYERS * STEPS_PER_LAYER) & (phase == 0))
    def _qkv_step():
        a = a_ref[...]
        u = jnp.dot(a, h_scr[...], preferred_element_type=jnp.float32)
        ones_col = jnp.ones((N_NODES, 8), jnp.float32)
        deg = jnp.dot(a, ones_col, preferred_element_type=jnp.float32)[:, :1]
        hn = lax.dot_general(u, w_ref[0], (((1,), (1,)), ((), ())),
                             preferred_element_type=jnp.float32) + deg * b_ref[0]
        qkv_scr[pl.ds(row, BM), :] = lax.dot_general(
            hn, inw_ref[...], (((1,), (1,)), ((), ())),
            preferred_element_type=jnp.float32) + inb_ref[...]

    @pl.when((g < LAYERS * STEPS_PER_LAYER) & (phase == 1))
    def _attn_step():
        kv_all = qkv_scr[:, HIDDEN:]
        qkv_q = qkv_scr[pl.ds(row, BM), :]
        ones_kcol = jnp.ones((N_NODES, 1), jnp.float32)
        outs = []
        for hh in range(HEADS):
            lo = hh * HEAD_DIM
            q = qkv_q[:, lo:lo + HEAD_DIM]
            k = kv_all[:, lo:lo + HEAD_DIM]
            v = kv_all[:, HIDDEN + lo:HIDDEN + lo + HEAD_DIM]
            sc = lax.dot_general(q, k, (((1,), (1,)), ((), ())),
                                 preferred_element_type=jnp.float32) * scale
            m = jnp.max(sc, axis=1, keepdims=True)
            e = jnp.exp(sc - m)
            nv = jnp.dot(e, jnp.concatenate([v, ones_kcol], axis=1),
                         preferred_element_type=jnp.float32)
            outs.append(nv[:, :HEAD_DIM] / nv[:, HEAD_DIM:HEAD_DIM + 1])
        att = jnp.concatenate(outs, axis=1)
        att = lax.dot_general(att, ow_ref[...], (((1,), (1,)), ((), ())),
                              preferred_element_type=jnp.float32) + ob_ref[...]
        h_scr[pl.ds(row, BM), :] = jnp.maximum(att + h_scr[pl.ds(row, BM), :], 0.0)

    @pl.when(g == N_STEPS - 1)
    def _proj_step():
        z = jnp.dot(h_scr[...], pw_ref[...],
                    preferred_element_type=jnp.float32)[:, :1] + pb_ref[0, 0]
        out_ref[...] = jax.nn.sigmoid(z)


def _fused_layers(A, h0, Wstack, bstack, in_w, in_b, out_w, out_b, pw, pb):
    cmax = LAYERS * STEPS_PER_LAYER

    def a_map(g):
        qkv_phase = ((g // GRID) % 2 == 0) & (g < cmax)
        return (jnp.where(qkv_phase, g % GRID, GRID - 1), 0)

    def w_map(g):
        return (jnp.minimum(g // STEPS_PER_LAYER, LAYERS - 1), 0, 0)

    const2 = lambda g: (0, 0)
    const3 = lambda g: (0, 0, 0)
    return pl.pallas_call(
        _fused_body,
        grid=(N_STEPS,),
        in_specs=[
            pl.BlockSpec((BM, N_NODES), a_map),
            pl.BlockSpec((N_NODES, HIDDEN), const2),
            pl.BlockSpec((1, HIDDEN, HIDDEN), w_map),
            pl.BlockSpec((1, 1, HIDDEN), w_map),
            pl.BlockSpec((3 * HIDDEN, HIDDEN), const2),
            pl.BlockSpec((1, 3 * HIDDEN), const2),
            pl.BlockSpec((HIDDEN, HIDDEN), const2),
            pl.BlockSpec((1, HIDDEN), const2),
            pl.BlockSpec((HIDDEN, 8), const2),
            pl.BlockSpec((1, 1), const2),
        ],
        out_specs=pl.BlockSpec((N_NODES, 1), const2),
        out_shape=jax.ShapeDtypeStruct((N_NODES, 1), jnp.float32),
        scratch_shapes=[
            pltpu.VMEM((N_NODES, 3 * HIDDEN), jnp.float32),
            pltpu.VMEM((N_NODES, HIDDEN), jnp.float32),
        ],
    )(A, h0, Wstack, bstack, in_w, in_b, out_w, out_b, pw, pb)


def kernel(node_embeddings, W0, b0, W1, b1, W2, b2, in_proj_w, in_proj_b,
           out_proj_w, out_proj_b, proj_w, proj_b, edge_index):
    a_flat = _make_build_adjacency()(edge_index.reshape(-1))
    A = a_flat.reshape(N_NODES, N_NODES)
    Wstack = jnp.stack((W0, W1, W2))
    bstack = jnp.stack((b0, b1, b2)).reshape(LAYERS, 1, HIDDEN)
    pw_pad = jnp.concatenate(
        [proj_w.reshape(HIDDEN, 1), jnp.zeros((HIDDEN, 7), jnp.float32)], axis=1)
    return _fused_layers(A, node_embeddings, Wstack, bstack, in_proj_w,
                         in_proj_b.reshape(1, 3 * HIDDEN), out_proj_w,
                         out_proj_b.reshape(1, HIDDEN), pw_pad,
                         proj_b.reshape(1, 1))
